# baseline (speedup 1.0000x reference)
import functools

import jax
import jax.numpy as jnp
from jax import lax
from jax.experimental import pallas as pl
from jax.experimental.pallas import tpu as pltpu
from jax.experimental.pallas import tpu_sc as plsc

_B, _N, _D = 128, 8192, 64
_L = 16
_NCH = _N // _L
_NW = 32
_RPW = _B // _NW
_DSL = 2
_NSL = _D // _DSL
_NPAIR = _NSL // 2


def _kvec(v):
    d = v * jnp.float32(_N) - jnp.float32(0.5)
    k = d.astype(jnp.int32) + 1
    k = jnp.where(d < jnp.float32(0.0), 0, k)
    return jnp.minimum(k, _N)


def _resample_call(st, cs, weight, maskf):
    mesh = plsc.VectorSubcoreMesh(core_axis_name="c", subcore_axis_name="s")

    @functools.partial(
        pl.kernel,
        out_type=(
            jax.ShapeDtypeStruct((_B, _D, _N), jnp.float32),
            jax.ShapeDtypeStruct((_B, _N), jnp.float32),
        ),
        mesh=mesh,
        scratch_types=[
            pltpu.VMEM((_N,), jnp.float32),
            pltpu.VMEM((_N,), jnp.float32),
            pltpu.VMEM((_N,), jnp.int32),
            pltpu.VMEM((_B,), jnp.float32),
            pltpu.VMEM((_NCH,), jnp.float32),
            pltpu.VMEM((_DSL, _N), jnp.float32),
            pltpu.VMEM((_DSL, _N), jnp.float32),
            pltpu.VMEM((_DSL, _N), jnp.float32),
            pltpu.VMEM((_DSL, _N), jnp.float32),
            pltpu.VMEM((_DSL, _N), jnp.float32),
            pltpu.VMEM((_DSL, _N), jnp.float32),
            pltpu.SemaphoreType.DMA,
            pltpu.SemaphoreType.DMA,
            pltpu.SemaphoreType.DMA,
            pltpu.SemaphoreType.DMA,
            pltpu.SemaphoreType.DMA,
            pltpu.SemaphoreType.DMA,
        ],
        compiler_params=pltpu.CompilerParams(needs_layout_passes=False),
    )
    def k(st_hbm, cs_hbm, w_hbm, m_hbm, outs_hbm, outw_hbm,
          cs_buf, a_buf, idx_buf, m_all, bmax,
          in_a, in_b, out_a, out_b, out_c, out_d,
          sia, sib, soa, sob, soc, sod):
        wid = lax.axis_index("s") * 2 + lax.axis_index("c")
        lane = lax.iota(jnp.int32, _L)
        zero16f = jnp.zeros((_L,), jnp.float32)
        invn = jnp.full((_L,), 1.0 / _N, jnp.float32)

        pltpu.sync_copy(m_hbm, m_all)

        @plsc.parallel_loop(0, _N, _L, unroll=4)
        def _zl(off):
            a_buf[pl.ds(off, _L)] = zero16f

        def _win(j, buf, sem):
            pltpu.make_async_copy(
                st_hbm.at[0, pl.ds(0, _DSL)], buf, sem).wait()

        def _wout(r, buf, sem):
            pltpu.make_async_copy(
                buf, outs_hbm.at[r, pl.ds(0, _DSL)], sem).wait()

        for kk in range(_RPW):
            r = wid + _NW * kk
            base = r * _N
            mch = m_all[pl.ds((r // _L) * _L, _L)]
            mval = jnp.max(jnp.where(lane == r % _L, mch, jnp.float32(0.0)))
            do_rs = mval != jnp.float32(0.0)

            @pl.when(do_rs)
            def _masked():
                pltpu.sync_copy(cs_hbm.at[r], cs_buf)

                @plsc.parallel_loop(0, _N, _L, unroll=4)
                def _p1(off):
                    glob = off + lane
                    v = cs_buf[pl.ds(off, _L)]
                    gi = jnp.maximum(glob - 1, 0)
                    vm1 = plsc.load_gather(cs_buf, [gi])
                    kcur = _kvec(v)
                    kcur = jnp.where(glob == _N - 1, _N, kcur)
                    kprev = _kvec(vm1)
                    kprev = jnp.where(glob == 0, 0, kprev)
                    mw = kcur > kprev
                    pos = jnp.minimum(kprev, _N - 1)
                    plsc.store_scatter(
                        a_buf, [pos], (base + glob).astype(jnp.float32),
                        mask=mw)

                @plsc.parallel_loop(0, _N, _L, unroll=4)
                def _wfill(off):
                    cs_buf[pl.ds(off, _L)] = invn
                pltpu.sync_copy(cs_buf, outw_hbm.at[r])

                basef = zero16f + base.astype(jnp.float32)

                @plsc.parallel_loop(0, _N, _L, unroll=4)
                def _p2a(off):
                    c16 = lane * 0 + off // _L
                    v = a_buf[pl.ds(off, _L)]
                    sc = plsc.cummax(v)
                    a_buf[pl.ds(off, _L)] = sc
                    cmx = jnp.max(sc)
                    plsc.store_scatter(
                        bmax, [c16], zero16f + cmx, mask=lane == 0)

                def p2b(c, m):
                    off = c * _L
                    v = bmax[pl.ds(off, _L)]
                    sc = jnp.maximum(plsc.cummax(v), m)
                    bmax[pl.ds(off, _L)] = sc
                    return jnp.max(sc)
                lax.fori_loop(0, _NCH // _L, p2b, base.astype(jnp.float32))

                @plsc.parallel_loop(0, _N, _L, unroll=4)
                def _p2c(off):
                    c16 = lane * 0 + off // _L
                    sc = a_buf[pl.ds(off, _L)]
                    pm1 = plsc.load_gather(bmax, [jnp.maximum(c16 - 1, 0)])
                    exc = jnp.where(c16 == 0, basef, pm1)
                    f = jnp.maximum(sc, exc)
                    idx_buf[pl.ds(off, _L)] = f.astype(jnp.int32) - base

                pltpu.async_copy(st_hbm.at[r, pl.ds(0, _DSL)], in_a, sia)
                pltpu.async_copy(st_hbm.at[r, pl.ds(_DSL, _DSL)], in_b, sib)

                def _gather_into(oa, ob):
                    @plsc.parallel_loop(0, _N, _L, unroll=8)
                    def _gath(off):
                        idx16 = idx_buf[pl.ds(off, _L)]
                        for dr in range(_DSL):
                            di = jnp.full((_L,), dr, jnp.int32)
                            oa[dr, pl.ds(off, _L)] = plsc.load_gather(
                                in_a, [di, idx16])
                            ob[dr, pl.ds(off, _L)] = plsc.load_gather(
                                in_b, [di, idx16])

                def quad(q, carry):
                    d0 = 4 * q * _DSL
                    _win(0, in_a, sia)
                    _win(1, in_b, sib)

                    @pl.when(q > 0)
                    def _drain_ab():
                        _wout(r, out_a, soa)
                        _wout(r, out_b, sob)
                    _gather_into(out_a, out_b)
                    pltpu.async_copy(
                        out_a, outs_hbm.at[r, pl.ds(d0, _DSL)], soa)
                    pltpu.async_copy(
                        out_b, outs_hbm.at[r, pl.ds(d0 + _DSL, _DSL)], sob)
                    pltpu.async_copy(
                        st_hbm.at[r, pl.ds(d0 + 2 * _DSL, _DSL)], in_a, sia)
                    pltpu.async_copy(
                        st_hbm.at[r, pl.ds(d0 + 3 * _DSL, _DSL)], in_b, sib)

                    _win(0, in_a, sia)
                    _win(1, in_b, sib)

                    @pl.when(q > 0)
                    def _drain_cd():
                        _wout(r, out_c, soc)
                        _wout(r, out_d, sod)
                    _gather_into(out_c, out_d)
                    pltpu.async_copy(
                        out_c, outs_hbm.at[r, pl.ds(d0 + 2 * _DSL, _DSL)],
                        soc)
                    pltpu.async_copy(
                        out_d, outs_hbm.at[r, pl.ds(d0 + 3 * _DSL, _DSL)],
                        sod)

                    @pl.when(q < _NPAIR // 2 - 1)
                    def _pre_next():
                        pltpu.async_copy(
                            st_hbm.at[r, pl.ds(d0 + 4 * _DSL, _DSL)],
                            in_a, sia)
                        pltpu.async_copy(
                            st_hbm.at[r, pl.ds(d0 + 5 * _DSL, _DSL)],
                            in_b, sib)
                    return carry
                lax.fori_loop(0, _NPAIR // 2, quad, 0)
                _wout(r, out_a, soa)
                _wout(r, out_b, sob)
                _wout(r, out_c, soc)
                _wout(r, out_d, sod)

            @pl.when(jnp.logical_not(do_rs))
            def _passthrough():
                pltpu.sync_copy(w_hbm.at[r], cs_buf)
                pltpu.sync_copy(cs_buf, outw_hbm.at[r])
                bufs = (in_a, in_b, out_a, out_b)
                sems = (sia, sib, soa, sob)
                for j in range(4):
                    pltpu.async_copy(
                        st_hbm.at[r, pl.ds(j * _DSL, _DSL)], bufs[j], sems[j])

                def cquad(q, carry):
                    d0 = 4 * q * _DSL
                    for j in range(4):
                        pltpu.make_async_copy(
                            st_hbm.at[r, pl.ds(0, _DSL)],
                            bufs[j], sems[j]).wait()
                        pltpu.async_copy(
                            bufs[j],
                            outs_hbm.at[r, pl.ds(d0 + j * _DSL, _DSL)],
                            sems[j])
                    for j in range(4):
                        pltpu.make_async_copy(
                            bufs[j], outs_hbm.at[r, pl.ds(0, _DSL)],
                            sems[j]).wait()

                        @pl.when(q < _NSL // 4 - 1)
                        def _pre():
                            pltpu.async_copy(
                                st_hbm.at[r,
                                          pl.ds(d0 + (4 + j) * _DSL, _DSL)],
                                bufs[j], sems[j])
                    return carry
                lax.fori_loop(0, _NSL // 4, cquad, 0)

    return k(st, cs, weight, maskf)


def kernel(state, weight):
    b, n, d = state.shape
    s = jnp.sum(weight, axis=1, keepdims=True)
    w = weight / s
    ess = 1.0 / jnp.sum(w * w, axis=1)
    mask = ess < 0.5 * n
    cs = jnp.cumsum(w, axis=1)
    st = jnp.transpose(state, (0, 2, 1))
    outs, outw = _resample_call(st, cs, weight, mask.astype(jnp.float32))
    return jnp.transpose(outs, (0, 2, 1)), outw


# speedup vs baseline: 1.0118x; 1.0118x over previous
import functools

import jax
import jax.numpy as jnp
from jax import lax
from jax.experimental import pallas as pl
from jax.experimental.pallas import tpu as pltpu
from jax.experimental.pallas import tpu_sc as plsc

_B, _N, _D = 128, 8192, 64
_L = 16
_NCH = _N // _L
_NW = 32
_RPW = _B // _NW
_DSL = 2
_SLW = _DSL * _N
_NSL = _D // _DSL
_NPAIR = _NSL // 2


def _kvec(v):
    d = v * jnp.float32(_N) - jnp.float32(0.5)
    k = d.astype(jnp.int32) + 1
    k = jnp.where(d < jnp.float32(0.0), 0, k)
    return jnp.minimum(k, _N)


def _resample_call(st, cs, weight, maskf):
    mesh = plsc.VectorSubcoreMesh(core_axis_name="c", subcore_axis_name="s")

    @functools.partial(
        pl.kernel,
        out_type=(
            jax.ShapeDtypeStruct((_B, _D, _N), jnp.float32),
            jax.ShapeDtypeStruct((_B, _N), jnp.float32),
        ),
        mesh=mesh,
        scratch_types=[
            pltpu.VMEM((_N,), jnp.float32),
            pltpu.VMEM((_N,), jnp.float32),
            pltpu.VMEM((_N,), jnp.float32),
            pltpu.VMEM((_N,), jnp.float32),
            pltpu.VMEM((_N,), jnp.int32),
            pltpu.VMEM((_B,), jnp.float32),
            pltpu.VMEM((_NCH,), jnp.float32),
            pltpu.VMEM((_DSL, _N), jnp.float32),
            pltpu.VMEM((_DSL, _N), jnp.float32),
            pltpu.VMEM((_DSL, _N), jnp.float32),
            pltpu.VMEM((_DSL, _N), jnp.float32),
            pltpu.SemaphoreType.DMA,
            pltpu.SemaphoreType.DMA,
            pltpu.SemaphoreType.DMA,
            pltpu.SemaphoreType.DMA,
        ],
        compiler_params=pltpu.CompilerParams(needs_layout_passes=False),
    )
    def k(st_hbm, cs_hbm, w_hbm, m_hbm, outs_hbm, outw_hbm,
          cs_buf, w_buf, wconst, a_buf, idx_buf, m_all, bmax,
          in_a, in_b, out_a, out_b, sia, sib, soa, sob):
        wid = lax.axis_index("s") * 2 + lax.axis_index("c")
        lane = lax.iota(jnp.int32, _L)
        zero16f = jnp.zeros((_L,), jnp.float32)
        invn = jnp.full((_L,), 1.0 / _N, jnp.float32)

        pltpu.sync_copy(m_hbm, m_all)

        @plsc.parallel_loop(0, _N, _L, unroll=4)
        def _zl(off):
            a_buf[pl.ds(off, _L)] = zero16f
            wconst[pl.ds(off, _L)] = invn

        for kk in range(_RPW):
            r = wid + _NW * kk
            base = r * _N
            mch = m_all[pl.ds((r // _L) * _L, _L)]
            mval = jnp.max(jnp.where(lane == r % _L, mch, jnp.float32(0.0)))
            do_rs = mval != jnp.float32(0.0)

            @pl.when(do_rs)
            def _masked():
                pltpu.sync_copy(cs_hbm.at[r], cs_buf)
                pltpu.sync_copy(wconst, outw_hbm.at[r])

                @plsc.parallel_loop(0, _N, _L, unroll=4)
                def _p1(off):
                    glob = off + lane
                    v = cs_buf[pl.ds(off, _L)]
                    gi = jnp.maximum(glob - 1, 0)
                    vm1 = plsc.load_gather(cs_buf, [gi])
                    kcur = _kvec(v)
                    kcur = jnp.where(glob == _N - 1, _N, kcur)
                    kprev = _kvec(vm1)
                    kprev = jnp.where(glob == 0, 0, kprev)
                    mw = kcur > kprev
                    pos = jnp.minimum(kprev, _N - 1)
                    plsc.store_scatter(
                        a_buf, [pos], (base + glob).astype(jnp.float32),
                        mask=mw)

                basef = zero16f + base.astype(jnp.float32)

                @plsc.parallel_loop(0, _N, _L, unroll=4)
                def _p2a(off):
                    c16 = lane * 0 + off // _L
                    v = a_buf[pl.ds(off, _L)]
                    sc = plsc.cummax(v)
                    w_buf[pl.ds(off, _L)] = sc
                    cmx = jnp.max(sc)
                    plsc.store_scatter(
                        bmax, [c16], zero16f + cmx, mask=lane == 0)

                def p2b(c, m):
                    off = c * _L
                    v = bmax[pl.ds(off, _L)]
                    sc = jnp.maximum(plsc.cummax(v), m)
                    bmax[pl.ds(off, _L)] = sc
                    return jnp.max(sc)
                lax.fori_loop(0, _NCH // _L, p2b, base.astype(jnp.float32))

                @plsc.parallel_loop(0, _N, _L, unroll=4)
                def _p2c(off):
                    c16 = lane * 0 + off // _L
                    sc = w_buf[pl.ds(off, _L)]
                    pm1 = plsc.load_gather(bmax, [jnp.maximum(c16 - 1, 0)])
                    exc = jnp.where(c16 == 0, basef, pm1)
                    f = jnp.maximum(sc, exc)
                    idx_buf[pl.ds(off, _L)] = f.astype(jnp.int32) - base

                pltpu.async_copy(st_hbm.at[r, pl.ds(0, _DSL)], in_a, sia)
                pltpu.async_copy(st_hbm.at[r, pl.ds(_DSL, _DSL)], in_b, sib)

                def pair(i, carry):
                    d0 = 2 * i * _DSL
                    pltpu.make_async_copy(
                        st_hbm.at[r, pl.ds(0, _DSL)], in_a, sia).wait()
                    pltpu.make_async_copy(
                        st_hbm.at[r, pl.ds(0, _DSL)], in_b, sib).wait()

                    @pl.when(i > 0)
                    def _drain_outs():
                        pltpu.make_async_copy(
                            out_a, outs_hbm.at[r, pl.ds(0, _DSL)], soa).wait()
                        pltpu.make_async_copy(
                            out_b, outs_hbm.at[r, pl.ds(0, _DSL)], sob).wait()

                    @plsc.parallel_loop(0, _N, _L, unroll=4)
                    def _gath(off):
                        idx16 = idx_buf[pl.ds(off, _L)]
                        for dr in range(_DSL):
                            di = jnp.full((_L,), dr, jnp.int32)
                            out_a[dr, pl.ds(off, _L)] = plsc.load_gather(
                                in_a, [di, idx16])
                            out_b[dr, pl.ds(off, _L)] = plsc.load_gather(
                                in_b, [di, idx16])

                    pltpu.async_copy(
                        out_a, outs_hbm.at[r, pl.ds(d0, _DSL)], soa)
                    pltpu.async_copy(
                        out_b, outs_hbm.at[r, pl.ds(d0 + _DSL, _DSL)], sob)

                    @pl.when(i < _NPAIR - 1)
                    def _prefetch():
                        pltpu.async_copy(
                            st_hbm.at[r, pl.ds(d0 + 2 * _DSL, _DSL)],
                            in_a, sia)
                        pltpu.async_copy(
                            st_hbm.at[r, pl.ds(d0 + 3 * _DSL, _DSL)],
                            in_b, sib)
                    return carry
                lax.fori_loop(0, _NPAIR, pair, 0)
                pltpu.make_async_copy(
                    out_a, outs_hbm.at[r, pl.ds(0, _DSL)], soa).wait()
                pltpu.make_async_copy(
                    out_b, outs_hbm.at[r, pl.ds(0, _DSL)], sob).wait()

            @pl.when(jnp.logical_not(do_rs))
            def _passthrough():
                pltpu.sync_copy(w_hbm.at[r], w_buf)
                pltpu.sync_copy(w_buf, outw_hbm.at[r])
                pltpu.async_copy(st_hbm.at[r, pl.ds(0, _DSL)], in_a, sia)
                pltpu.async_copy(st_hbm.at[r, pl.ds(_DSL, _DSL)], in_b, sib)

                def cpair(i, carry):
                    d0 = 2 * i * _DSL
                    pltpu.make_async_copy(
                        st_hbm.at[r, pl.ds(0, _DSL)], in_a, sia).wait()
                    pltpu.make_async_copy(
                        st_hbm.at[r, pl.ds(0, _DSL)], in_b, sib).wait()
                    pltpu.async_copy(
                        in_a, outs_hbm.at[r, pl.ds(d0, _DSL)], soa)
                    pltpu.async_copy(
                        in_b, outs_hbm.at[r, pl.ds(d0 + _DSL, _DSL)], sob)
                    pltpu.make_async_copy(
                        in_a, outs_hbm.at[r, pl.ds(0, _DSL)], soa).wait()
                    pltpu.make_async_copy(
                        in_b, outs_hbm.at[r, pl.ds(0, _DSL)], sob).wait()

                    @pl.when(i < _NPAIR - 1)
                    def _prefetch2():
                        pltpu.async_copy(
                            st_hbm.at[r, pl.ds(d0 + 2 * _DSL, _DSL)],
                            in_a, sia)
                        pltpu.async_copy(
                            st_hbm.at[r, pl.ds(d0 + 3 * _DSL, _DSL)],
                            in_b, sib)
                    return carry
                lax.fori_loop(0, _NPAIR, cpair, 0)

    return k(st, cs, weight, maskf)


def kernel(state, weight):
    b, n, d = state.shape
    s = jnp.sum(weight, axis=1, keepdims=True)
    w = weight / s
    ess = 1.0 / jnp.sum(w * w, axis=1)
    mask = ess < 0.5 * n
    cs = jnp.cumsum(w, axis=1)
    st = jnp.transpose(state, (0, 2, 1))
    outs, outw = _resample_call(st, cs, weight, mask.astype(jnp.float32))
    return jnp.transpose(outs, (0, 2, 1)), outw
